# baseline (speedup 1.0000x reference)
import jax
import jax.numpy as jnp
from jax.experimental import pallas as pl
from jax.experimental.pallas import tpu as pltpu

_T = 1024
_D = 1024
_L = 3072
_NMEL = 100
_KCH = 6
_LN_EPS = 1e-5
_F32 = jnp.float32
_BF16 = jnp.bfloat16


def _tap_perm():
    r = jax.lax.broadcasted_iota(jnp.int32, (192, 192), 0)
    c = jax.lax.broadcasted_iota(jnp.int32, (192, 192), 1)
    t = c // 64
    i = c % 64
    return (r == 3 * i + t).astype(_BF16)


def _gelu(x):
    return 0.5 * x * (1.0 + jax.lax.erf(x * 0.7071067811865476))


def _conv_quad(xq_f32, w_ref, ci, qi, brow):
    m = jax.lax.dot_general(xq_f32.astype(_BF16), w_ref[ci, qi],
                            (((1,), (1,)), ((), ())),
                            preferred_element_type=_F32)
    m0 = m[:, 0:256]
    m1 = m[:, 256:512]
    m2 = m[:, 512:768]
    z = jnp.zeros((1, 256), _F32)
    y = m1 + brow
    y = y + jnp.concatenate([z, m0[:-1]], axis=0)
    y = y + jnp.concatenate([m2[1:], z], axis=0)
    return y


def _main_body(ll_ref, win_ref, wd_ref, bd_ref, p_ref, mask_ref,
               w1f_ref, w2f_ref, bin_ref, cb1_ref, cb2_ref,
               lng_ref, lnb_ref, wmel_ref, bmel_ref, o_ref,
               acc_ref, a_acc, winb_s, q_s, cw_s, p_s, sig_s, w1b_s, w2b_s):
    b = pl.program_id(0)
    k = pl.program_id(1)

    kq = jnp.minimum(k, 3)

    @pl.when(b == 0)
    def _():
        p_s[kq] = p_ref[...].astype(_BF16)
        sig_s[kq] = jax.nn.sigmoid(mask_ref[...]).astype(_BF16)
        w1b_s[kq] = w1f_ref[...].astype(_BF16)
        w2b_s[kq] = w2f_ref[...].astype(_BF16)
        wb = win_ref[...].astype(_BF16)
        winb_s[k] = wb
        part_a = jax.lax.dot_general(wb, wd_ref[...].astype(_BF16),
                                     (((1,), (1,)), ((), ())),
                                     preferred_element_type=_F32)

        @pl.when(k == 0)
        def _():
            a_acc[...] = part_a

        @pl.when(k > 0)
        def _():
            a_acc[...] = a_acc[...] + part_a

    part = jax.lax.dot_general(ll_ref[0].astype(_BF16), winb_s[k],
                               (((1,), (1,)), ((), ())),
                               preferred_element_type=_F32)

    @pl.when(k < _KCH - 1)
    def _():
        @pl.when(k == 0)
        def _():
            acc_ref[...] = part

        @pl.when(k > 0)
        def _():
            acc_ref[...] = acc_ref[...] + part

    @pl.when((b == 0) & (k == _KCH - 1))
    def _():
        perm = _tap_perm()
        cw_s[...] = jnp.zeros(cw_s.shape, _BF16)
        for c, wsc in enumerate((w1b_s, w2b_s)):
            wf = jnp.concatenate([wsc[0], wsc[1], wsc[2], wsc[3]], axis=0)
            wt = jnp.dot(wf, perm,
                         preferred_element_type=_F32)
            for t in range(3):
                for q in range(4):
                    for gl in range(4):
                        r0 = 256 * t + 64 * gl
                        s0 = 256 * q + 64 * gl
                        cw_s[c, q, r0:r0 + 64, 64 * gl:64 * (gl + 1)] = (
                            wt[s0:s0 + 64, 64 * t:64 * (t + 1)].astype(_BF16))
        a = a_acc[...] + bd_ref[...]
        p = jnp.concatenate([p_s[0], p_s[1], p_s[2], p_s[3]], axis=0)
        sig = jnp.concatenate([sig_s[0], sig_s[1], sig_s[2], sig_s[3]],
                              axis=0)
        t1 = jax.lax.dot_general(p, a.astype(_BF16), (((0,), (0,)), ((), ())),
                                 preferred_element_type=_F32)
        t2 = jnp.dot(t1.astype(_BF16), p, preferred_element_type=_F32)
        a_acc[...] = t2 * sig.astype(_F32)
        u = jax.lax.dot_general(p, a_acc[...].astype(_BF16),
                                (((1,), (1,)), ((), ())),
                                preferred_element_type=_F32)
        qm = jax.lax.dot_general(u.astype(_BF16), p, (((1,), (1,)), ((), ())),
                                 preferred_element_type=_F32)
        q_s[...] = qm.astype(_BF16)

    @pl.when(k == _KCH - 1)
    def _():
        xb = (acc_ref[...] + part + bin_ref[...]).astype(_BF16)
        acc_ref[...] = jnp.dot(xb, q_s[...], preferred_element_type=_F32)
        for qi in range(4):
            s = slice(qi * 256, (qi + 1) * 256)
            h = _conv_quad(acc_ref[:, s], cw_s, 0, qi, cb1_ref[:, s])
            h = _gelu(h)
            h = _conv_quad(h, cw_s, 1, qi, cb2_ref[:, s])
            acc_ref[:, s] = h
        x2 = acc_ref[...]
        mu = jnp.mean(x2, axis=1, keepdims=True)
        d = x2 - mu
        var = jnp.mean(d * d, axis=1, keepdims=True)
        xn = d * jax.lax.rsqrt(var + _LN_EPS) * lng_ref[...] + lnb_ref[...]
        mel = jax.lax.dot_general(wmel_ref[...].astype(_BF16),
                                  xn.astype(_BF16),
                                  (((1,), (1,)), ((), ())),
                                  preferred_element_type=_F32)
        o_ref[0] = mel + bmel_ref[...]


def kernel(llama_embeddings, timesteps, W_in, b_in, P, spectral_mask, Wd, bd,
           conv1_w, conv1_b, conv2_w, conv2_b, ln_g, ln_b, Wmel, bmel):
    B, T, L = llama_embeddings.shape
    kc = L // _KCH
    last = _KCH - 1

    def _wk_map(b, k):
        return (0, jnp.where(b == 0, k, last))

    out = pl.pallas_call(
        _main_body,
        out_shape=jax.ShapeDtypeStruct((B, _NMEL, T), _F32),
        grid=(B, _KCH),
        in_specs=[
            pl.BlockSpec((1, T, kc), lambda b, k: (b, 0, k)),
            pl.BlockSpec((_D, kc), _wk_map),
            pl.BlockSpec((_D, kc), _wk_map),
            pl.BlockSpec((1, _D), lambda b, k: (0, 0)),
            pl.BlockSpec((256, _D),
                         lambda b, k: (jnp.where(b == 0, jnp.minimum(k, 3), 3), 0)),
            pl.BlockSpec((256, _D),
                         lambda b, k: (jnp.where(b == 0, jnp.minimum(k, 3), 3), 0)),
            pl.BlockSpec((256, 192),
                         lambda b, k: (jnp.where(b == 0, jnp.minimum(k, 3), 3), 0)),
            pl.BlockSpec((256, 192),
                         lambda b, k: (jnp.where(b == 0, jnp.minimum(k, 3), 3), 0)),
            pl.BlockSpec((1, _D), lambda b, k: (0, 0)),
            pl.BlockSpec((1, _D), lambda b, k: (0, 0)),
            pl.BlockSpec((1, _D), lambda b, k: (0, 0)),
            pl.BlockSpec((1, _D), lambda b, k: (0, 0)),
            pl.BlockSpec((1, _D), lambda b, k: (0, 0)),
            pl.BlockSpec((_NMEL, _D), lambda b, k: (0, 0)),
            pl.BlockSpec((_NMEL, 1), lambda b, k: (0, 0)),
        ],
        out_specs=pl.BlockSpec((1, _NMEL, T), lambda b, k: (b, 0, 0)),
        scratch_shapes=[
            pltpu.VMEM((_T, _D), _F32),
            pltpu.VMEM((_D, _D), _F32),
            pltpu.VMEM((_KCH, _D, kc), _BF16),
            pltpu.VMEM((_D, _D), _BF16),
            pltpu.VMEM((2, 4, 768, 256), _BF16),
            pltpu.VMEM((4, 256, _D), _BF16),
            pltpu.VMEM((4, 256, _D), _BF16),
            pltpu.VMEM((4, 256, 192), _BF16),
            pltpu.VMEM((4, 256, 192), _BF16),
        ],
        compiler_params=pltpu.CompilerParams(
            dimension_semantics=("arbitrary", "arbitrary"),
            vmem_limit_bytes=52 * 1024 * 1024),
        name="csm_fused",
    )(llama_embeddings, W_in, Wd, bd.reshape(1, _D), P, spectral_mask,
      conv1_w.reshape(_D, 192), conv2_w.reshape(_D, 192),
      b_in.reshape(1, _D), conv1_b.reshape(1, _D), conv2_b.reshape(1, _D),
      ln_g.reshape(1, _D), ln_b.reshape(1, _D), Wmel,
      bmel.reshape(_NMEL, 1))
    return out


# speedup vs baseline: 1.0878x; 1.0878x over previous
import jax
import jax.numpy as jnp
from jax.experimental import pallas as pl
from jax.experimental.pallas import tpu as pltpu

_T = 1024
_D = 1024
_L = 3072
_NMEL = 100
_KCH = 2
_LN_EPS = 1e-5
_F32 = jnp.float32
_BF16 = jnp.bfloat16


_PCH = 4


def _tap_perm():
    r = jax.lax.broadcasted_iota(jnp.int32, (192, 192), 0)
    c = jax.lax.broadcasted_iota(jnp.int32, (192, 192), 1)
    t = c // 64
    i = c % 64
    return (r == 3 * i + t).astype(_BF16)


def _prep_body(win_ref, wd_ref, bd_ref, p_ref, mask_ref, wmel_ref,
               w1f_ref, w2f_ref,
               q_ref, winb_ref, wmelb_ref, cw_ref, a_acc):
    k = pl.program_id(0)
    win_bf = win_ref[...].astype(_BF16)
    winb_ref[...] = win_bf
    part = jax.lax.dot_general(win_bf, wd_ref[...].astype(_BF16),
                               (((1,), (1,)), ((), ())),
                               preferred_element_type=_F32)

    @pl.when(k == 0)
    def _():
        a_acc[...] = part

    @pl.when(k > 0)
    def _():
        a_acc[...] = a_acc[...] + part

    @pl.when(k == _PCH - 1)
    def _():
        wmelb_ref[...] = wmel_ref[...].astype(_BF16)
        perm = _tap_perm()
        cw_ref[...] = jnp.zeros(cw_ref.shape, _BF16)
        for c, wref in enumerate((w1f_ref, w2f_ref)):
            wt = jnp.dot(wref[...].astype(_BF16), perm,
                         preferred_element_type=_F32)
            for t in range(3):
                for q in range(4):
                    for gl in range(4):
                        r0 = 256 * t + 64 * gl
                        s0 = 256 * q + 64 * gl
                        cw_ref[c, q, r0:r0 + 64, 64 * gl:64 * (gl + 1)] = (
                            wt[s0:s0 + 64, 64 * t:64 * (t + 1)].astype(_BF16))
        a = a_acc[...] + bd_ref[...]
        p = p_ref[...].astype(_BF16)
        t1 = jax.lax.dot_general(p, a.astype(_BF16),
                                 (((0,), (0,)), ((), ())),
                                 preferred_element_type=_F32)
        t2 = jnp.dot(t1.astype(_BF16), p, preferred_element_type=_F32)
        masked = t2 * jax.nn.sigmoid(mask_ref[...])
        u = jax.lax.dot_general(p, masked.astype(_BF16),
                                (((1,), (1,)), ((), ())),
                                preferred_element_type=_F32)
        qm = jax.lax.dot_general(u.astype(_BF16), p, (((1,), (1,)), ((), ())),
                                 preferred_element_type=_F32)
        q_ref[...] = qm.astype(_BF16)


def _gelu(x):
    return 0.5 * x * (1.0 + jax.lax.erf(x * 0.7071067811865476))


def _conv_quad(xq_f32, w_ref, ci, qi, brow):
    m = jax.lax.dot_general(xq_f32.astype(_BF16), w_ref[ci, qi],
                            (((1,), (1,)), ((), ())),
                            preferred_element_type=_F32)
    m0 = m[:, 0:256]
    m1 = m[:, 256:512]
    m2 = m[:, 512:768]
    z = jnp.zeros((1, 256), _F32)
    y = m1 + brow
    y = y + jnp.concatenate([z, m0[:-1]], axis=0)
    y = y + jnp.concatenate([m2[1:], z], axis=0)
    return y


def _main_body(ll_ref, win_ref, bin_ref, q_ref, cw_ref, cb1_ref, cb2_ref,
               lng_ref, lnb_ref, wmel_ref, bmel_ref, o_ref, acc_ref, winb_s):
    b = pl.program_id(0)
    k = pl.program_id(1)

    @pl.when(b == 0)
    def _():
        winb_s[k] = win_ref[...]

    part = jax.lax.dot_general(ll_ref[0].astype(_BF16), winb_s[k],
                               (((1,), (1,)), ((), ())),
                               preferred_element_type=_F32)

    @pl.when(k < _KCH - 1)
    def _():
        @pl.when(k == 0)
        def _():
            acc_ref[...] = part

        @pl.when(k > 0)
        def _():
            acc_ref[...] = acc_ref[...] + part

    @pl.when(k == _KCH - 1)
    def _():
        xb = (acc_ref[...] + part + bin_ref[...]).astype(_BF16)
        fused = jnp.dot(xb, q_ref[...], preferred_element_type=_F32)
        quads = []
        for qi in range(4):
            s = slice(qi * 256, (qi + 1) * 256)
            h = _conv_quad(fused[:, s], cw_ref, 0, qi, cb1_ref[:, s])
            h = _gelu(h)
            h = _conv_quad(h, cw_ref, 1, qi, cb2_ref[:, s])
            quads.append(h)
        x2 = jnp.concatenate(quads, axis=1)
        mu = jnp.mean(x2, axis=1, keepdims=True)
        d = x2 - mu
        var = jnp.mean(d * d, axis=1, keepdims=True)
        xn = d * jax.lax.rsqrt(var + _LN_EPS) * lng_ref[...] + lnb_ref[...]
        mel = jax.lax.dot_general(wmel_ref[...], xn.astype(_BF16),
                                  (((1,), (1,)), ((), ())),
                                  preferred_element_type=_F32)
        o_ref[0] = mel + bmel_ref[...]


def kernel(llama_embeddings, timesteps, W_in, b_in, P, spectral_mask, Wd, bd,
           conv1_w, conv1_b, conv2_w, conv2_b, ln_g, ln_b, Wmel, bmel):
    B, T, L = llama_embeddings.shape

    pch = _L // _PCH
    q, win_bf, wmel_bf, cw = pl.pallas_call(
        _prep_body,
        out_shape=(jax.ShapeDtypeStruct((_D, _D), _BF16),
                   jax.ShapeDtypeStruct((_D, _L), _BF16),
                   jax.ShapeDtypeStruct((_NMEL, _D), _BF16),
                   jax.ShapeDtypeStruct((2, 4, 768, 256), _BF16)),
        grid=(_PCH,),
        in_specs=[
            pl.BlockSpec((_D, pch), lambda k: (0, k)),
            pl.BlockSpec((_D, pch), lambda k: (0, k)),
            pl.BlockSpec((1, _D), lambda k: (0, 0)),
            pl.BlockSpec((_D, _D), lambda k: (0, 0)),
            pl.BlockSpec((_D, _D), lambda k: (0, 0)),
            pl.BlockSpec((_NMEL, _D), lambda k: (0, 0)),
            pl.BlockSpec((_D, 192), lambda k: (0, 0)),
            pl.BlockSpec((_D, 192), lambda k: (0, 0)),
        ],
        out_specs=(
            pl.BlockSpec((_D, _D), lambda k: (0, 0)),
            pl.BlockSpec((_D, pch), lambda k: (0, k)),
            pl.BlockSpec((_NMEL, _D), lambda k: (0, 0)),
            pl.BlockSpec((2, 4, 768, 256), lambda k: (0, 0, 0, 0)),
        ),
        scratch_shapes=[pltpu.VMEM((_D, _D), _F32)],
        compiler_params=pltpu.CompilerParams(
            dimension_semantics=("arbitrary",),
            vmem_limit_bytes=48 * 1024 * 1024),
        name="csm_prep_q",
    )(W_in, Wd, bd.reshape(1, _D), P, spectral_mask, Wmel,
      conv1_w.reshape(_D, 192), conv2_w.reshape(_D, 192))


    kc = L // _KCH
    out = pl.pallas_call(
        _main_body,
        out_shape=jax.ShapeDtypeStruct((B, _NMEL, T), _F32),
        grid=(B, _KCH),
        in_specs=[
            pl.BlockSpec((1, T, kc), lambda b, k: (b, 0, k)),
            pl.BlockSpec((_D, kc), lambda b, k: (0, jnp.where(b == 0, k, _KCH - 1))),
            pl.BlockSpec((1, _D), lambda b, k: (0, 0)),
            pl.BlockSpec((_D, _D), lambda b, k: (0, 0)),
            pl.BlockSpec((2, 4, 768, 256), lambda b, k: (0, 0, 0, 0)),
            pl.BlockSpec((1, _D), lambda b, k: (0, 0)),
            pl.BlockSpec((1, _D), lambda b, k: (0, 0)),
            pl.BlockSpec((1, _D), lambda b, k: (0, 0)),
            pl.BlockSpec((1, _D), lambda b, k: (0, 0)),
            pl.BlockSpec((_NMEL, _D), lambda b, k: (0, 0)),
            pl.BlockSpec((_NMEL, 1), lambda b, k: (0, 0)),
        ],
        out_specs=pl.BlockSpec((1, _NMEL, T), lambda b, k: (b, 0, 0)),
        scratch_shapes=[pltpu.VMEM((_T, _D), _F32),
                        pltpu.VMEM((_KCH, _D, kc), _BF16)],
        compiler_params=pltpu.CompilerParams(
            dimension_semantics=("arbitrary", "arbitrary"),
            vmem_limit_bytes=48 * 1024 * 1024),
        name="csm_main",
    )(llama_embeddings, win_bf, b_in.reshape(1, _D), q, cw,
      conv1_b.reshape(1, _D), conv2_b.reshape(1, _D),
      ln_g.reshape(1, _D), ln_b.reshape(1, _D), wmel_bf,
      bmel.reshape(_NMEL, 1))
    return out
